# Initial kernel scaffold; baseline (speedup 1.0000x reference)
import jax
import jax.numpy as jnp
from jax.experimental import pallas as pl


def kernel(x, emb, W, b):
    raise NotImplementedError("write your pallas kernel here")


# speedup vs baseline: 11.0474x; 11.0474x over previous
import functools

import jax
import jax.numpy as jnp
from jax import lax
from jax.experimental import pallas as pl
from jax.experimental.pallas import tpu as pltpu
from jax.experimental.pallas import tpu_sc as plsc

B = 16384
HIST = 200
D = 32
CLS = 100

NC = 2
NS = 16
NW = NC * NS
BPW = B // NW
CH = 16
NCHUNK = BPW // CH
IDX_N = CH * HIST
GSL = 128
NG = IDX_N // GSL

_mesh = plsc.VectorSubcoreMesh(core_axis_name="c", subcore_axis_name="s")


@functools.partial(
    pl.kernel,
    mesh=_mesh,
    out_type=jax.ShapeDtypeStruct((B, D), jnp.float32),
    compiler_params=pltpu.CompilerParams(use_tc_tiling_on_sc=False),
    scratch_types=[
        pltpu.VMEM((IDX_N,), jnp.int32),
        pltpu.VMEM((IDX_N, D), jnp.float32),
        pltpu.VMEM((CH, D), jnp.float32),
        pltpu.SemaphoreType.DMA,
    ],
)
def _sc_bag_sum(x_hbm, emb_hbm, out_hbm, idx_v, rows_v, sum_v, sem):
    cid = lax.axis_index("c")
    sid = lax.axis_index("s")
    wid = sid * NC + cid
    base = wid * BPW

    def chunk_body(c, carry):
        off = base + c * CH
        pltpu.sync_copy(x_hbm.at[pl.ds(off * HIST, IDX_N)], idx_v)
        copies = []
        for g in range(NG):
            copies.append(
                pltpu.async_copy(
                    emb_hbm.at[idx_v.at[pl.ds(g * GSL, GSL)]],
                    rows_v.at[pl.ds(g * GSL, GSL)],
                    sem,
                ))
        for cp in copies:
            cp.wait()

        def sample_body(s, carry2):
            def row_body(j, accs):
                a0, a1 = accs
                r = s * HIST + j
                a0 = a0 + rows_v[r, pl.ds(0, 16)]
                a1 = a1 + rows_v[r, pl.ds(16, 16)]
                return (a0, a1)

            zero = jnp.zeros((16,), jnp.float32)
            a0, a1 = lax.fori_loop(0, HIST, row_body, (zero, zero))
            sum_v[s, pl.ds(0, 16)] = a0
            sum_v[s, pl.ds(16, 16)] = a1
            return carry2

        lax.fori_loop(0, CH, sample_body, 0)
        pltpu.sync_copy(sum_v, out_hbm.at[pl.ds(off, CH)])
        return carry

    lax.fori_loop(0, NCHUNK, chunk_body, 0)


_TC_BLK = 2048


def _tc_head_body(x_ref, sum_ref, w_ref, b_ref, o_ref):
    cnt = jnp.sum((x_ref[...] != 0).astype(jnp.float32), axis=1,
                  keepdims=True)
    mean = sum_ref[...] * (1.0 / jnp.maximum(cnt, 1.0))
    o_ref[...] = (
        jnp.dot(mean, w_ref[...], preferred_element_type=jnp.float32)
        + b_ref[...])


_tc_head = pl.pallas_call(
    _tc_head_body,
    grid=(B // _TC_BLK,),
    in_specs=[
        pl.BlockSpec((_TC_BLK, HIST), lambda i: (i, 0)),
        pl.BlockSpec((_TC_BLK, D), lambda i: (i, 0)),
        pl.BlockSpec((D, CLS), lambda i: (0, 0)),
        pl.BlockSpec((1, CLS), lambda i: (0, 0)),
    ],
    out_specs=pl.BlockSpec((_TC_BLK, CLS), lambda i: (i, 0)),
    out_shape=jax.ShapeDtypeStruct((B, CLS), jnp.float32),
)


def kernel(x, emb, W, b):
    x = x.astype(jnp.int32)
    summed = _sc_bag_sum(x.reshape(B * HIST), emb)
    return _tc_head(x, summed, W, b.reshape(1, CLS))


# speedup vs baseline: 16.2665x; 1.4724x over previous
import functools

import jax
import jax.numpy as jnp
from jax import lax
from jax.experimental import pallas as pl
from jax.experimental.pallas import tpu as pltpu
from jax.experimental.pallas import tpu_sc as plsc

B = 16384
HIST = 200
D = 32
CLS = 100

NC = 2
NS = 16
NW = NC * NS
BPW = B // NW
CH = 8
NCHUNK = BPW // CH
IDX_N = CH * HIST
GSL = 80
NG = IDX_N // GSL
UNROLL = 8

_mesh = plsc.VectorSubcoreMesh(core_axis_name="c", subcore_axis_name="s")


@functools.partial(
    pl.kernel,
    mesh=_mesh,
    out_type=jax.ShapeDtypeStruct((B, D), jnp.float32),
    compiler_params=pltpu.CompilerParams(use_tc_tiling_on_sc=False),
    scratch_types=[
        pltpu.VMEM((2, IDX_N), jnp.int32),
        pltpu.VMEM((IDX_N, D), jnp.float32),
        pltpu.VMEM((IDX_N, D), jnp.float32),
        pltpu.VMEM((CH, D), jnp.float32),
        pltpu.SemaphoreType.DMA,
        pltpu.SemaphoreType.DMA,
    ],
)
def _sc_bag_sum(x_hbm, emb_hbm, out_hbm, idx_v, rows0_v, rows1_v, sum_v,
                sem0, sem1):
    cid = lax.axis_index("c")
    sid = lax.axis_index("s")
    wid = sid * NC + cid
    base = wid * BPW
    rows_bufs = (rows0_v, rows1_v)
    sems = (sem0, sem1)

    def fire(c, b):
        off = base + c * CH
        pltpu.sync_copy(x_hbm.at[pl.ds(off * HIST, IDX_N)], idx_v.at[b])
        for g in range(NG):
            pltpu.async_copy(
                emb_hbm.at[idx_v.at[b, pl.ds(g * GSL, GSL)]],
                rows_bufs[b].at[pl.ds(g * GSL, GSL)],
                sems[b],
            )

    def drain(b):
        for g in range(NG):
            pltpu.make_async_copy(
                emb_hbm.at[idx_v.at[b, pl.ds(g * GSL, GSL)]],
                rows_bufs[b].at[pl.ds(g * GSL, GSL)],
                sems[b],
            ).wait()

    def consume(c, b):
        rows_v = rows_bufs[b]
        for s in range(CH):
            def row_body(j, accs):
                a0, a1 = accs
                r = s * HIST + j * UNROLL
                for u in range(UNROLL):
                    a0 = a0 + rows_v[r + u, pl.ds(0, 16)]
                    a1 = a1 + rows_v[r + u, pl.ds(16, 16)]
                return (a0, a1)

            zero = jnp.zeros((16,), jnp.float32)
            a0, a1 = lax.fori_loop(0, HIST // UNROLL, row_body, (zero, zero))
            sum_v[s, pl.ds(0, 16)] = a0
            sum_v[s, pl.ds(16, 16)] = a1
        off = base + c * CH
        pltpu.sync_copy(sum_v, out_hbm.at[pl.ds(off, CH)])

    fire(0, 0)

    def pair_body(c2, carry):
        for b in range(2):
            c = c2 + b

            @pl.when(c + 1 < NCHUNK)
            def _():
                fire(c + 1, 1 - b)

            drain(b)
            consume(c, b)
        return carry

    lax.fori_loop(0, NCHUNK // 2, lambda i, cr: pair_body(i * 2, cr), 0)


_TC_BLK = 2048


def _tc_head_body(x_ref, sum_ref, w_ref, b_ref, o_ref):
    cnt = jnp.sum((x_ref[...] != 0).astype(jnp.float32), axis=1,
                  keepdims=True)
    mean = sum_ref[...] * (1.0 / jnp.maximum(cnt, 1.0))
    o_ref[...] = (
        jnp.dot(mean, w_ref[...], preferred_element_type=jnp.float32)
        + b_ref[...])


_tc_head = pl.pallas_call(
    _tc_head_body,
    grid=(B // _TC_BLK,),
    in_specs=[
        pl.BlockSpec((_TC_BLK, HIST), lambda i: (i, 0)),
        pl.BlockSpec((_TC_BLK, D), lambda i: (i, 0)),
        pl.BlockSpec((D, CLS), lambda i: (0, 0)),
        pl.BlockSpec((1, CLS), lambda i: (0, 0)),
    ],
    out_specs=pl.BlockSpec((_TC_BLK, CLS), lambda i: (i, 0)),
    out_shape=jax.ShapeDtypeStruct((B, CLS), jnp.float32),
)


def kernel(x, emb, W, b):
    x = x.astype(jnp.int32)
    summed = _sc_bag_sum(x.reshape(B * HIST), emb)
    return _tc_head(x, summed, W, b.reshape(1, CLS))


# speedup vs baseline: 16.2743x; 1.0005x over previous
import functools

import jax
import jax.numpy as jnp
from jax import lax
from jax.experimental import pallas as pl
from jax.experimental.pallas import tpu as pltpu
from jax.experimental.pallas import tpu_sc as plsc

B = 16384
HIST = 200
D = 32
CLS = 100

NC = 2
NS = 16
NW = NC * NS
BPW = B // NW
CH = 8
NCHUNK = BPW // CH
IDX_N = CH * HIST
GS0 = 104
GS1 = HIST - GS0
UNROLL = 8

_mesh = plsc.VectorSubcoreMesh(core_axis_name="c", subcore_axis_name="s")


@functools.partial(
    pl.kernel,
    mesh=_mesh,
    out_type=jax.ShapeDtypeStruct((B, D), jnp.float32),
    compiler_params=pltpu.CompilerParams(use_tc_tiling_on_sc=False),
    scratch_types=[
        pltpu.VMEM((2, CH, HIST), jnp.int32),
        pltpu.VMEM((IDX_N, D), jnp.float32),
        pltpu.VMEM((IDX_N, D), jnp.float32),
        pltpu.VMEM((CH, D), jnp.float32),
        pltpu.SemaphoreType.DMA,
        pltpu.SemaphoreType.DMA,
    ],
)
def _sc_bag_sum(x_hbm, emb_hbm, out_hbm, idx_v, rows0_v, rows1_v, sum_v,
                sem0, sem1):
    cid = lax.axis_index("c")
    sid = lax.axis_index("s")
    wid = sid * NC + cid
    base = wid * BPW
    rows_bufs = (rows0_v, rows1_v)
    sems = (sem0, sem1)

    def gather_list(c, b):
        out = []
        for s in range(CH):
            out.append((idx_v.at[b, s, pl.ds(0, GS0)],
                        rows_bufs[b].at[pl.ds(s * HIST, GS0)]))
            out.append((idx_v.at[b, s, pl.ds(GS0, GS1)],
                        rows_bufs[b].at[pl.ds(s * HIST + GS0, GS1)]))
        return out

    def fire(c, b):
        off = base + c * CH
        pltpu.sync_copy(x_hbm.at[pl.ds(off, CH), :], idx_v.at[b])
        for isl, rsl in gather_list(c, b):
            pltpu.async_copy(emb_hbm.at[isl], rsl, sems[b])

    def drain(b):
        for isl, rsl in gather_list(0, b):
            pltpu.make_async_copy(emb_hbm.at[isl], rsl, sems[b]).wait()

    def consume(c, b):
        rows_v = rows_bufs[b]
        for s in range(CH):
            def row_body(j, accs):
                a0, a1 = accs
                r = s * HIST + j * UNROLL
                for u in range(UNROLL):
                    a0 = a0 + rows_v[r + u, pl.ds(0, 16)]
                    a1 = a1 + rows_v[r + u, pl.ds(16, 16)]
                return (a0, a1)

            zero = jnp.zeros((16,), jnp.float32)
            a0, a1 = lax.fori_loop(0, HIST // UNROLL, row_body, (zero, zero))
            sum_v[s, pl.ds(0, 16)] = a0
            sum_v[s, pl.ds(16, 16)] = a1
        off = base + c * CH
        pltpu.sync_copy(sum_v, out_hbm.at[pl.ds(off, CH)])

    fire(0, 0)

    def pair_body(c2, carry):
        for b in range(2):
            c = c2 + b

            @pl.when(c + 1 < NCHUNK)
            def _():
                fire(c + 1, 1 - b)

            drain(b)
            consume(c, b)
        return carry

    lax.fori_loop(0, NCHUNK // 2, lambda i, cr: pair_body(i * 2, cr), 0)


_TC_BLK = 2048


def _tc_head_body(x_ref, sum_ref, w_ref, b_ref, o_ref):
    cnt = jnp.sum((x_ref[...] != 0).astype(jnp.float32), axis=1,
                  keepdims=True)
    mean = sum_ref[...] * (1.0 / jnp.maximum(cnt, 1.0))
    o_ref[...] = (
        jnp.dot(mean, w_ref[...], preferred_element_type=jnp.float32)
        + b_ref[...])


_tc_head = pl.pallas_call(
    _tc_head_body,
    grid=(B // _TC_BLK,),
    in_specs=[
        pl.BlockSpec((_TC_BLK, HIST), lambda i: (i, 0)),
        pl.BlockSpec((_TC_BLK, D), lambda i: (i, 0)),
        pl.BlockSpec((D, CLS), lambda i: (0, 0)),
        pl.BlockSpec((1, CLS), lambda i: (0, 0)),
    ],
    out_specs=pl.BlockSpec((_TC_BLK, CLS), lambda i: (i, 0)),
    out_shape=jax.ShapeDtypeStruct((B, CLS), jnp.float32),
)


def kernel(x, emb, W, b):
    x = x.astype(jnp.int32)
    summed = _sc_bag_sum(x, emb)
    return _tc_head(x, summed, W, b.reshape(1, CLS))
